# Initial kernel scaffold; baseline (speedup 1.0000x reference)
import jax
import jax.numpy as jnp
from jax.experimental import pallas as pl


def kernel(y_pred, mask, centers):
    raise NotImplementedError("write your pallas kernel here")


# speedup vs baseline: 1.8945x; 1.8945x over previous
import jax
import jax.numpy as jnp
from jax.experimental import pallas as pl
from jax.experimental.pallas import tpu as pltpu


def _assign_body(mt_ref, x_ref, c_ref, o_ref, *, T):
    t = pl.program_id(0) % T
    scale = jnp.where(mt_ref[t] == 0.0, 1.0, 0.0).astype(jnp.float32)
    x = x_ref[...] * scale
    c = c_ref[...]
    logits = 2.0 * jax.lax.dot_general(
        x, c, (((1,), (1,)), ((), ())),
        preferred_element_type=jnp.float32)
    c2 = jnp.sum(c * c, axis=1)
    logits = logits - c2[None, :]
    m = jnp.max(logits, axis=-1, keepdims=True)
    e = jnp.exp(logits - m)
    o_ref[...] = e / jnp.sum(e, axis=-1, keepdims=True)


def kernel(y_pred, mask, centers):
    B, T, H, D = y_pred.shape
    K = centers.shape[0]
    N = B * T

    x2d = y_pred.reshape(N, H, D).reshape(N * H, D)
    masktime = mask[0, :, 0, 0]

    import functools
    body = functools.partial(_assign_body, T=T)

    out = pl.pallas_call(
        body,
        grid=(N,),
        in_specs=[
            pl.BlockSpec(memory_space=pltpu.SMEM),
            pl.BlockSpec((H, D), lambda i: (i, 0)),
            pl.BlockSpec((K, D), lambda i: (0, 0)),
        ],
        out_specs=pl.BlockSpec((H, K), lambda i: (i, 0)),
        out_shape=jax.ShapeDtypeStruct((N * H, K), jnp.float32),
        compiler_params=pltpu.CompilerParams(
            dimension_semantics=("arbitrary",)),
    )(masktime, x2d, centers)

    return out.reshape(B, T, H, K)
